# baseline (speedup 1.0000x reference)
import functools
import jax
import jax.numpy as jnp
from jax import lax
from jax.experimental import pallas as pl
from jax.experimental.pallas import tpu as pltpu
from jax.experimental.pallas import tpu_sc as plsc

VOCAB = 1000000
D = 64
SCALE = 8.0

NC = 2
NS = 16
NW = NC * NS

NROW = 4096
SEQ = 50
SEQ_PAD = 64
R_PER_W = NROW // NW
NBUF = 4
N_OUTER = R_PER_W // NBUF


def _make_kernel():
    mesh = plsc.VectorSubcoreMesh(core_axis_name="c", subcore_axis_name="s")

    @functools.partial(
        pl.kernel,
        mesh=mesh,
        out_type=jax.ShapeDtypeStruct((NROW, SEQ, D), jnp.float32),
        scratch_types=[
            pltpu.VMEM((R_PER_W, SEQ_PAD), jnp.int32),
            pltpu.VMEM((NBUF, SEQ_PAD, D), jnp.float32),
            pltpu.VMEM((NBUF, SEQ, D), jnp.float32),
        ]
        + [pltpu.SemaphoreType.DMA] * (2 * NBUF),
        compiler_params=pltpu.CompilerParams(use_tc_tiling_on_sc=False),
    )
    def emb_kernel(x_hbm, lut_hbm, out_hbm, xbuf, gbuf, wbuf, *sems):
        gsems = sems[:NBUF]
        wsems = sems[NBUF:]
        wid = lax.axis_index("s") * NC + lax.axis_index("c")
        row0 = wid * R_PER_W
        pltpu.sync_copy(x_hbm.at[pl.ds(row0, R_PER_W)], xbuf)

        def gather(i, b):
            return pltpu.async_copy(
                lut_hbm.at[xbuf.at[i]], gbuf.at[b], gsems[b])

        for b in range(NBUF):
            gather(b, b)

        def outer(g, carry):
            for b in range(NBUF):
                i = g * NBUF + b
                pltpu.make_async_copy(
                    lut_hbm.at[xbuf.at[i]], gbuf.at[b], gsems[b]).wait()
                @pl.when(g > 0)
                def _():
                    pltpu.make_async_copy(
                        wbuf.at[b], out_hbm.at[row0], wsems[b]).wait()
                for r in range(SEQ):
                    for j in range(D // 16):
                        sl = pl.ds(j * 16, 16)
                        wbuf[b, r, sl] = gbuf[b, r, sl] * SCALE
                @pl.when(i + NBUF < R_PER_W)
                def _():
                    gather(i + NBUF, b)
                pltpu.async_copy(
                    wbuf.at[b], out_hbm.at[row0 + i], wsems[b])
            return carry

        lax.fori_loop(0, N_OUTER, outer, 0)
        for b in range(NBUF):
            pltpu.make_async_copy(
                wbuf.at[b], out_hbm.at[row0], wsems[b]).wait()

    return emb_kernel


_emb = _make_kernel()


@jax.jit
def kernel(x, lut):
    xp = jnp.pad(x.astype(jnp.int32), ((0, 0), (0, SEQ_PAD - SEQ)))
    return _emb(xp, lut)


# speedup vs baseline: 2.4641x; 2.4641x over previous
import functools
import jax
import jax.numpy as jnp
from jax import lax
from jax.experimental import pallas as pl
from jax.experimental.pallas import tpu as pltpu
from jax.experimental.pallas import tpu_sc as plsc

VOCAB = 1000000
D = 64
SCALE = 8.0

NC = 2
NS = 16
NW = NC * NS

NROW = 4096
SEQ = 50
LANES = 128
R_PER_W = NROW // NW
NBUF = 4
N_OUTER = R_PER_W // NBUF


SEQ8 = 56


def _widen_body(x_ref, o_ref):
    o_ref[:, :SEQ] = x_ref[...]
    o_ref[:, SEQ:SEQ8] = x_ref[:, : SEQ8 - SEQ]


_widen = pl.pallas_call(
    _widen_body,
    out_shape=jax.ShapeDtypeStruct((NROW, LANES), jnp.int32),
    grid=(8,),
    in_specs=[pl.BlockSpec((NROW // 8, SEQ), lambda i: (i, 0))],
    out_specs=pl.BlockSpec((NROW // 8, LANES), lambda i: (i, 0)),
)


def _make_kernel():
    mesh = plsc.VectorSubcoreMesh(core_axis_name="c", subcore_axis_name="s")

    @functools.partial(
        pl.kernel,
        mesh=mesh,
        out_type=jax.ShapeDtypeStruct((NROW, SEQ, D), jnp.float32),
        scratch_types=[
            pltpu.VMEM((R_PER_W, LANES), jnp.int32),
            pltpu.VMEM((NBUF, SEQ8, D), jnp.float32),
            pltpu.VMEM((NBUF, SEQ, D), jnp.float32),
        ]
        + [pltpu.SemaphoreType.DMA] * (2 * NBUF),
        compiler_params=pltpu.CompilerParams(use_tc_tiling_on_sc=False),
    )
    def emb_kernel(x_hbm, lut_hbm, out_hbm, xbuf, gbuf, wbuf, *sems):
        gsems = sems[:NBUF]
        wsems = sems[NBUF:]
        wid = lax.axis_index("s") * NC + lax.axis_index("c")
        row0 = wid * R_PER_W
        pltpu.sync_copy(x_hbm.at[pl.ds(row0, R_PER_W)], xbuf)

        def gather(i, b):
            return pltpu.async_copy(
                lut_hbm.at[xbuf.at[i, pl.ds(0, SEQ8)]], gbuf.at[b], gsems[b])

        for b in range(NBUF):
            gather(b, b)

        def outer(g, carry):
            for b in range(NBUF):
                i = g * NBUF + b
                pltpu.make_async_copy(
                    lut_hbm.at[xbuf.at[i, pl.ds(0, SEQ8)]],
                    gbuf.at[b], gsems[b]).wait()
                @pl.when(g > 0)
                def _():
                    pltpu.make_async_copy(
                        wbuf.at[b], out_hbm.at[row0], wsems[b]).wait()
                for r in range(SEQ):
                    for j in range(D // 16):
                        sl = pl.ds(j * 16, 16)
                        wbuf[b, r, sl] = gbuf[b, r, sl] * SCALE
                @pl.when(i + NBUF < R_PER_W)
                def _():
                    gather(i + NBUF, b)
                pltpu.async_copy(
                    wbuf.at[b], out_hbm.at[row0 + i], wsems[b])
            return carry

        lax.fori_loop(0, N_OUTER, outer, 0)
        for b in range(NBUF):
            pltpu.make_async_copy(
                wbuf.at[b], out_hbm.at[row0], wsems[b]).wait()

    return emb_kernel


_emb = _make_kernel()


@jax.jit
def kernel(x, lut):
    xw = _widen(x.astype(jnp.int32))
    return _emb(xw, lut)


# speedup vs baseline: 2.4689x; 1.0019x over previous
import functools
import jax
import jax.numpy as jnp
from jax import lax
from jax.experimental import pallas as pl
from jax.experimental.pallas import tpu as pltpu
from jax.experimental.pallas import tpu_sc as plsc

VOCAB = 1000000
D = 64
SCALE = 8.0

NC = 2
NS = 16
NW = NC * NS

NROW = 4096
SEQ = 50
LANES = 128
R_PER_W = NROW // NW
NBUF = 4
N_OUTER = R_PER_W // NBUF


SEQ8 = 56


def _widen_body(x_ref, o_ref):
    o_ref[:, :SEQ] = x_ref[...]
    o_ref[:, SEQ:SEQ8] = x_ref[:, : SEQ8 - SEQ]


_widen = pl.pallas_call(
    _widen_body,
    out_shape=jax.ShapeDtypeStruct((NROW, LANES), jnp.int32),
    grid=(8,),
    in_specs=[pl.BlockSpec((NROW // 8, SEQ), lambda i: (i, 0))],
    out_specs=pl.BlockSpec((NROW // 8, LANES), lambda i: (i, 0)),
)


def _make_kernel():
    mesh = plsc.VectorSubcoreMesh(core_axis_name="c", subcore_axis_name="s")

    @functools.partial(
        pl.kernel,
        mesh=mesh,
        out_type=jax.ShapeDtypeStruct((NROW, SEQ, D), jnp.float32),
        scratch_types=[
            pltpu.VMEM((R_PER_W * LANES,), jnp.int32),
            pltpu.VMEM((NBUF, SEQ8, D), jnp.float32),
            pltpu.VMEM((NBUF, SEQ, D), jnp.float32),
        ]
        + [pltpu.SemaphoreType.DMA] * (2 * NBUF),
        compiler_params=pltpu.CompilerParams(use_tc_tiling_on_sc=False),
    )
    def emb_kernel(x_hbm, lut_hbm, out_hbm, xbuf, gbuf, wbuf, *sems):
        gsems = sems[:NBUF]
        wsems = sems[NBUF:]
        wid = lax.axis_index("s") * NC + lax.axis_index("c")
        row0 = wid * R_PER_W
        pltpu.sync_copy(
            x_hbm.at[pl.ds(row0 * LANES, R_PER_W * LANES)], xbuf)

        def gather(i, b):
            return pltpu.async_copy(
                lut_hbm.at[xbuf.at[pl.ds(i * LANES, SEQ8)]],
                gbuf.at[b], gsems[b])

        for b in range(NBUF):
            gather(b, b)

        def outer(g, carry):
            for b in range(NBUF):
                i = g * NBUF + b
                pltpu.make_async_copy(
                    lut_hbm.at[xbuf.at[pl.ds(i * LANES, SEQ8)]],
                    gbuf.at[b], gsems[b]).wait()
                @pl.when(g > 0)
                def _():
                    pltpu.make_async_copy(
                        wbuf.at[b], out_hbm.at[row0], wsems[b]).wait()
                for r in range(SEQ):
                    for j in range(D // 16):
                        sl = pl.ds(j * 16, 16)
                        wbuf[b, r, sl] = gbuf[b, r, sl] * SCALE
                @pl.when(i + NBUF < R_PER_W)
                def _():
                    gather(i + NBUF, b)
                pltpu.async_copy(
                    wbuf.at[b], out_hbm.at[row0 + i], wsems[b])
            return carry

        lax.fori_loop(0, N_OUTER, outer, 0)
        for b in range(NBUF):
            pltpu.make_async_copy(
                wbuf.at[b], out_hbm.at[row0], wsems[b]).wait()

    return emb_kernel


_emb = _make_kernel()


@jax.jit
def kernel(x, lut):
    xw = _widen(x.astype(jnp.int32))
    return _emb(xw.reshape(-1), lut)


# speedup vs baseline: 2.4812x; 1.0050x over previous
import functools
import jax
import jax.numpy as jnp
from jax import lax
from jax.experimental import pallas as pl
from jax.experimental.pallas import tpu as pltpu
from jax.experimental.pallas import tpu_sc as plsc

VOCAB = 1000000
D = 64
SCALE = 8.0

NC = 2
NS = 16
NW = NC * NS

NROW = 4096
SEQ = 50
R_PER_W = NROW // NW
NBUF = 4
N_OUTER = R_PER_W // NBUF


def _make_kernel():
    mesh = plsc.VectorSubcoreMesh(core_axis_name="c", subcore_axis_name="s")

    @functools.partial(
        pl.kernel,
        mesh=mesh,
        out_type=jax.ShapeDtypeStruct((NROW, SEQ, D), jnp.float32),
        scratch_types=[
            pltpu.VMEM((R_PER_W, SEQ), jnp.int32),
            pltpu.VMEM((NBUF, SEQ, D), jnp.float32),
            pltpu.VMEM((NBUF, SEQ, D), jnp.float32),
        ]
        + [pltpu.SemaphoreType.DMA] * (2 * NBUF),
        compiler_params=pltpu.CompilerParams(use_tc_tiling_on_sc=False),
    )
    def emb_kernel(x_hbm, lut_hbm, out_hbm, xbuf, gbuf, wbuf, *sems):
        gsems = sems[:NBUF]
        wsems = sems[NBUF:]
        wid = lax.axis_index("s") * NC + lax.axis_index("c")
        row0 = wid * R_PER_W
        pltpu.sync_copy(x_hbm.at[pl.ds(row0, R_PER_W)], xbuf)

        def gather(i, b):
            return pltpu.async_copy(
                lut_hbm.at[xbuf.at[i]], gbuf.at[b], gsems[b])

        for b in range(NBUF):
            gather(b, b)

        def outer(g, carry):
            for b in range(NBUF):
                i = g * NBUF + b
                pltpu.make_async_copy(
                    lut_hbm.at[xbuf.at[i]], gbuf.at[b], gsems[b]).wait()
                @pl.when(g > 0)
                def _():
                    pltpu.make_async_copy(
                        wbuf.at[b], out_hbm.at[row0], wsems[b]).wait()
                for r in range(SEQ):
                    for j in range(D // 16):
                        sl = pl.ds(j * 16, 16)
                        wbuf[b, r, sl] = gbuf[b, r, sl] * SCALE
                @pl.when(i + NBUF < R_PER_W)
                def _():
                    gather(i + NBUF, b)
                pltpu.async_copy(
                    wbuf.at[b], out_hbm.at[row0 + i], wsems[b])
            return carry

        lax.fori_loop(0, N_OUTER, outer, 0)
        for b in range(NBUF):
            pltpu.make_async_copy(
                wbuf.at[b], out_hbm.at[row0], wsems[b]).wait()

    return emb_kernel


_emb = _make_kernel()


@jax.jit
def kernel(x, lut):
    return _emb(x.astype(jnp.int32), lut)
